# baseline (speedup 1.0000x reference)
import jax
import jax.numpy as jnp
from jax.experimental import pallas as pl

B = 512
NCAT = 16
NEXP = 64
EPC = 8
HIDDEN = 256

BB1 = 8
BB2 = 16
BB3 = 256


def _conv1_kernel(x_ref, w_ref, b_ref, o_ref):
    bb = x_ref.shape[0]
    x = x_ref[...].reshape(bb * 21 * 21, 192)
    y = jnp.dot(x, w_ref[...], preferred_element_type=jnp.float32)
    y = y.reshape(bb, 21, 21, 128)
    o = (y[:, 0:20, 0:20, 0:32] + y[:, 0:20, 1:21, 32:64]
         + y[:, 1:21, 0:20, 64:96] + y[:, 1:21, 1:21, 96:128])
    o = o + b_ref[...].reshape(1, 1, 1, 32)
    o_ref[...] = jnp.maximum(o, 0.0)


def _conv23_kernel(x_ref, w2_ref, b2_ref, w3_ref, b3_ref, f_ref):
    bb = x_ref.shape[0]
    x = x_ref[...].reshape(bb * 10 * 10, 128)
    y2 = jnp.dot(x, w2_ref[...], preferred_element_type=jnp.float32)
    y2 = y2.reshape(bb, 10, 10, 256)
    o2 = (y2[:, 0:9, 0:9, 0:64] + y2[:, 0:9, 1:10, 64:128]
          + y2[:, 1:10, 0:9, 128:192] + y2[:, 1:10, 1:10, 192:256])
    o2 = jnp.maximum(o2 + b2_ref[...].reshape(1, 1, 1, 64), 0.0)

    y3 = jnp.dot(o2.reshape(bb * 81, 64), w3_ref[...],
                 preferred_element_type=jnp.float32)
    y3 = y3.reshape(bb, 9, 9, 576)
    o3 = 0.0
    for kh in range(3):
        for kw in range(3):
            g = (kh * 3 + kw) * 64
            o3 = o3 + y3[:, kh:kh + 7, kw:kw + 7, g:g + 64]
    o3 = jnp.maximum(o3 + b3_ref[...].reshape(1, 1, 1, 64), 0.0)
    f_ref[...] = o3.reshape(bb, 7 * 7 * 64)


def _dense_kernel(f_ref, pw_ref, pb_ref, cw_ref, cb_ref, w1_ref, b1_ref,
                  w2_ref, b2_ref, oh_ref, hid_ref, log_ref, exp_ref):
    f = f_ref[...]
    hid = jnp.maximum(
        jnp.dot(f, pw_ref[...], preferred_element_type=jnp.float32)
        + pb_ref[...], 0.0)
    hid_ref[...] = hid
    logits = jnp.dot(hid, cw_ref[...], preferred_element_type=jnp.float32) \
        + cb_ref[...]
    log_ref[...] = logits
    m = jnp.max(logits, axis=-1, keepdims=True)
    e = jnp.exp(logits - m)
    probs = e / jnp.sum(e, axis=-1, keepdims=True)

    h1 = jnp.maximum(
        jnp.dot(hid, w1_ref[...], preferred_element_type=jnp.float32)
        + b1_ref[...], 0.0)
    z = jnp.dot(h1, w2_ref[...], preferred_element_type=jnp.float32) \
        + b2_ref[...]
    local = jax.nn.sigmoid(z)
    bb = f.shape[0]
    wts = jnp.broadcast_to(probs[:, :, None], (bb, NCAT, EPC))
    weighted = wts.reshape(bb, NCAT * EPC) * local
    exp_ref[...] = jnp.dot(weighted, oh_ref[...],
                           preferred_element_type=jnp.float32)


def kernel(obs, conv1_w, conv1_b, conv2_w, conv2_b, conv3_w, conv3_b,
           proj_w, proj_b, cat_w, cat_b, head_w1, head_b1, head_w2,
           head_b2, mapping):
    f32 = jnp.float32
    bsz = obs.shape[0]

    x1 = obs.reshape(bsz, 12, 21, 4, 21, 4).transpose(0, 2, 4, 1, 3, 5)
    x1 = x1.reshape(bsz, 21, 21, 192)
    w1a = conv1_w.reshape(32, 12, 2, 4, 2, 4).transpose(1, 3, 5, 2, 4, 0)
    w1a = w1a.reshape(192, 128)
    b1 = conv1_b.reshape(1, 32)

    w2a = conv2_w.reshape(64, 32, 2, 2, 2, 2).transpose(3, 5, 1, 2, 4, 0)
    w2a = w2a.reshape(128, 256)
    b2 = conv2_b.reshape(1, 64)

    w3a = conv3_w.transpose(1, 2, 3, 0).reshape(64, 576)
    b3 = conv3_b.reshape(1, 64)

    pw = proj_w.reshape(64, 7, 7, HIDDEN).transpose(1, 2, 0, 3)
    pw = pw.reshape(7 * 7 * 64, HIDDEN)
    pb = proj_b.reshape(1, HIDDEN)
    cb = cat_b.reshape(1, NCAT)

    wh1 = head_w1.transpose(1, 0, 2).reshape(HIDDEN, NCAT * (HIDDEN // 2))
    bh1 = head_b1.reshape(1, NCAT * (HIDDEN // 2))
    eye = jnp.eye(NCAT, dtype=f32)
    w2bd = (eye[:, None, :, None] * head_w2[:, :, None, :])
    w2bd = w2bd.reshape(NCAT * (HIDDEN // 2), NCAT * EPC)
    bh2 = head_b2.reshape(1, NCAT * EPC)

    onehot = (mapping.reshape(-1)[:, None]
              == jnp.arange(NEXP, dtype=jnp.int32)[None, :]).astype(f32)

    o1 = pl.pallas_call(
        _conv1_kernel,
        grid=(bsz // BB1,),
        in_specs=[
            pl.BlockSpec((BB1, 21, 21, 192), lambda i: (i, 0, 0, 0)),
            pl.BlockSpec((192, 128), lambda i: (0, 0)),
            pl.BlockSpec((1, 32), lambda i: (0, 0)),
        ],
        out_specs=pl.BlockSpec((BB1, 20, 20, 32), lambda i: (i, 0, 0, 0)),
        out_shape=jax.ShapeDtypeStruct((bsz, 20, 20, 32), f32),
    )(x1, w1a, b1)

    x2 = o1.reshape(bsz, 10, 2, 10, 2, 32).transpose(0, 1, 3, 2, 4, 5)
    x2 = x2.reshape(bsz, 10, 10, 128)

    feats = pl.pallas_call(
        _conv23_kernel,
        grid=(bsz // BB2,),
        in_specs=[
            pl.BlockSpec((BB2, 10, 10, 128), lambda i: (i, 0, 0, 0)),
            pl.BlockSpec((128, 256), lambda i: (0, 0)),
            pl.BlockSpec((1, 64), lambda i: (0, 0)),
            pl.BlockSpec((64, 576), lambda i: (0, 0)),
            pl.BlockSpec((1, 64), lambda i: (0, 0)),
        ],
        out_specs=pl.BlockSpec((BB2, 3136), lambda i: (i, 0)),
        out_shape=jax.ShapeDtypeStruct((bsz, 3136), f32),
    )(x2, w2a, b2, w3a, b3)

    hidden, logits, expert = pl.pallas_call(
        _dense_kernel,
        grid=(bsz // BB3,),
        in_specs=[
            pl.BlockSpec((BB3, 3136), lambda i: (i, 0)),
            pl.BlockSpec((3136, HIDDEN), lambda i: (0, 0)),
            pl.BlockSpec((1, HIDDEN), lambda i: (0, 0)),
            pl.BlockSpec((HIDDEN, NCAT), lambda i: (0, 0)),
            pl.BlockSpec((1, NCAT), lambda i: (0, 0)),
            pl.BlockSpec((HIDDEN, 2048), lambda i: (0, 0)),
            pl.BlockSpec((1, 2048), lambda i: (0, 0)),
            pl.BlockSpec((2048, 128), lambda i: (0, 0)),
            pl.BlockSpec((1, 128), lambda i: (0, 0)),
            pl.BlockSpec((128, NEXP), lambda i: (0, 0)),
        ],
        out_specs=[
            pl.BlockSpec((BB3, HIDDEN), lambda i: (i, 0)),
            pl.BlockSpec((BB3, NCAT), lambda i: (i, 0)),
            pl.BlockSpec((BB3, NEXP), lambda i: (i, 0)),
        ],
        out_shape=[
            jax.ShapeDtypeStruct((bsz, HIDDEN), f32),
            jax.ShapeDtypeStruct((bsz, NCAT), f32),
            jax.ShapeDtypeStruct((bsz, NEXP), f32),
        ],
    )(feats, pw, pb, cat_w, cb, wh1, bh1, w2bd, bh2, onehot)

    return (logits, expert, hidden)


# speedup vs baseline: 2.2659x; 2.2659x over previous
import jax
import jax.numpy as jnp
from jax.experimental import pallas as pl

B = 512
NCAT = 16
NEXP = 64
EPC = 8
HIDDEN = 256

BBC = 8
BB3 = 256


def _convs_kernel(x_ref, w1_ref, b1_ref, w2_ref, b2_ref, w3_ref, b3_ref,
                  f_ref):
    bb = x_ref.shape[2]
    bf16 = jnp.bfloat16
    x = x_ref[...].reshape(21 * 21 * bb, 192)
    y1 = jnp.dot(x, w1_ref[...], preferred_element_type=jnp.float32)
    y1 = y1.reshape(21, 21, bb, 128)
    o1 = (y1[0:20, 0:20, :, 0:32] + y1[0:20, 1:21, :, 32:64]
          + y1[1:21, 0:20, :, 64:96] + y1[1:21, 1:21, :, 96:128])
    o1 = jnp.maximum(o1 + b1_ref[...].reshape(1, 1, 1, 32), 0.0)
    o1r = o1.astype(bf16).reshape(10, 2, 10, 2, bb, 32)
    x2 = jnp.concatenate(
        [o1r[:, i, :, j] for i in range(2) for j in range(2)], axis=-1)
    y2 = jnp.dot(x2.reshape(100 * bb, 128), w2_ref[...],
                 preferred_element_type=jnp.float32)
    y2 = y2.reshape(10, 10, bb, 256)
    o2 = (y2[0:9, 0:9, :, 0:64] + y2[0:9, 1:10, :, 64:128]
          + y2[1:10, 0:9, :, 128:192] + y2[1:10, 1:10, :, 192:256])
    o2 = jnp.maximum(o2 + b2_ref[...].reshape(1, 1, 1, 64), 0.0)
    y3 = jnp.dot(o2.astype(bf16).reshape(81 * bb, 64), w3_ref[...],
                 preferred_element_type=jnp.float32)
    y3 = y3.reshape(9, 9, bb, 576)
    o3 = 0.0
    for kh in range(3):
        for kw in range(3):
            g = (kh * 3 + kw) * 64
            o3 = o3 + y3[kh:kh + 7, kw:kw + 7, :, g:g + 64]
    o3 = jnp.maximum(o3 + b3_ref[...].reshape(1, 1, 1, 64), 0.0)
    f_ref[...] = o3.astype(bf16).transpose(2, 0, 1, 3).reshape(bb, 3136)


def _dense_kernel(f_ref, pw_ref, pb_ref, cw_ref, cb_ref, w1_ref, b1_ref,
                  w2_ref, b2_ref, oh_ref, hid_ref, log_ref, exp_ref):
    f = f_ref[...]
    hid = jnp.maximum(
        jnp.dot(f, pw_ref[...], preferred_element_type=jnp.float32)
        + pb_ref[...], 0.0)
    hid_ref[...] = hid
    logits = jnp.dot(hid, cw_ref[...], preferred_element_type=jnp.float32) \
        + cb_ref[...]
    log_ref[...] = logits
    m = jnp.max(logits, axis=-1, keepdims=True)
    e = jnp.exp(logits - m)
    probs = e / jnp.sum(e, axis=-1, keepdims=True)

    h1 = jnp.maximum(
        jnp.dot(hid, w1_ref[...], preferred_element_type=jnp.float32)
        + b1_ref[...], 0.0)
    z = jnp.dot(h1, w2_ref[...], preferred_element_type=jnp.float32) \
        + b2_ref[...]
    local = jax.nn.sigmoid(z)
    bb = f.shape[0]
    wts = jnp.broadcast_to(probs[:, :, None], (bb, NCAT, EPC))
    weighted = wts.reshape(bb, NCAT * EPC) * local
    exp_ref[...] = jnp.dot(weighted, oh_ref[...],
                           preferred_element_type=jnp.float32)


def kernel(obs, conv1_w, conv1_b, conv2_w, conv2_b, conv3_w, conv3_b,
           proj_w, proj_b, cat_w, cat_b, head_w1, head_b1, head_w2,
           head_b2, mapping):
    f32 = jnp.float32
    bf16 = jnp.bfloat16
    bsz = obs.shape[0]

    x1 = obs.reshape(bsz, 12, 21, 4, 21, 4).transpose(2, 4, 0, 1, 3, 5)
    x1 = x1.reshape(21, 21, bsz, 192).astype(bf16)
    w1a = conv1_w.reshape(32, 12, 2, 4, 2, 4).transpose(1, 3, 5, 2, 4, 0)
    w1a = w1a.reshape(192, 128).astype(bf16)
    b1 = conv1_b.reshape(1, 32)

    w2a = conv2_w.reshape(64, 32, 2, 2, 2, 2).transpose(3, 5, 1, 2, 4, 0)
    w2a = w2a.reshape(128, 256).astype(bf16)
    b2 = conv2_b.reshape(1, 64)

    w3a = conv3_w.transpose(1, 2, 3, 0).reshape(64, 576).astype(bf16)
    b3 = conv3_b.reshape(1, 64)

    pw = proj_w.reshape(64, 7, 7, HIDDEN).transpose(1, 2, 0, 3)
    pw = pw.reshape(7 * 7 * 64, HIDDEN).astype(bf16)
    pb = proj_b.reshape(1, HIDDEN)
    cb = cat_b.reshape(1, NCAT)

    wh1 = head_w1.transpose(1, 0, 2).reshape(HIDDEN, NCAT * (HIDDEN // 2))
    bh1 = head_b1.reshape(1, NCAT * (HIDDEN // 2))
    eye = jnp.eye(NCAT, dtype=f32)
    w2bd = (eye[:, None, :, None] * head_w2[:, :, None, :])
    w2bd = w2bd.reshape(NCAT * (HIDDEN // 2), NCAT * EPC)
    bh2 = head_b2.reshape(1, NCAT * EPC)

    onehot = (mapping.reshape(-1)[:, None]
              == jnp.arange(NEXP, dtype=jnp.int32)[None, :]).astype(f32)

    feats = pl.pallas_call(
        _convs_kernel,
        grid=(bsz // BBC,),
        in_specs=[
            pl.BlockSpec((21, 21, BBC, 192), lambda i: (0, 0, i, 0)),
            pl.BlockSpec((192, 128), lambda i: (0, 0)),
            pl.BlockSpec((1, 32), lambda i: (0, 0)),
            pl.BlockSpec((128, 256), lambda i: (0, 0)),
            pl.BlockSpec((1, 64), lambda i: (0, 0)),
            pl.BlockSpec((64, 576), lambda i: (0, 0)),
            pl.BlockSpec((1, 64), lambda i: (0, 0)),
        ],
        out_specs=pl.BlockSpec((BBC, 3136), lambda i: (i, 0)),
        out_shape=jax.ShapeDtypeStruct((bsz, 3136), bf16),
    )(x1, w1a, b1, w2a, b2, w3a, b3)

    hidden, logits, expert = pl.pallas_call(
        _dense_kernel,
        grid=(bsz // BB3,),
        in_specs=[
            pl.BlockSpec((BB3, 3136), lambda i: (i, 0)),
            pl.BlockSpec((3136, HIDDEN), lambda i: (0, 0)),
            pl.BlockSpec((1, HIDDEN), lambda i: (0, 0)),
            pl.BlockSpec((HIDDEN, NCAT), lambda i: (0, 0)),
            pl.BlockSpec((1, NCAT), lambda i: (0, 0)),
            pl.BlockSpec((HIDDEN, 2048), lambda i: (0, 0)),
            pl.BlockSpec((1, 2048), lambda i: (0, 0)),
            pl.BlockSpec((2048, 128), lambda i: (0, 0)),
            pl.BlockSpec((1, 128), lambda i: (0, 0)),
            pl.BlockSpec((128, NEXP), lambda i: (0, 0)),
        ],
        out_specs=[
            pl.BlockSpec((BB3, HIDDEN), lambda i: (i, 0)),
            pl.BlockSpec((BB3, NCAT), lambda i: (i, 0)),
            pl.BlockSpec((BB3, NEXP), lambda i: (i, 0)),
        ],
        out_shape=[
            jax.ShapeDtypeStruct((bsz, HIDDEN), f32),
            jax.ShapeDtypeStruct((bsz, NCAT), f32),
            jax.ShapeDtypeStruct((bsz, NEXP), f32),
        ],
    )(feats, pw, pb, cat_w, cb, wh1, bh1, w2bd, bh2, onehot)

    return (logits, expert, hidden)
